# baseline (speedup 1.0000x reference)
import functools

import jax
import jax.numpy as jnp
from jax import lax
from jax.experimental import pallas as pl
from jax.experimental.pallas import tpu as pltpu
from jax.experimental.pallas import tpu_sc as plsc

N = 10000
E = 160000
NODE_DIM = 128
EDGE_DIM = 16
ATOM = 32
CONV = 128
NGRAPH = 64
NUM_EMBEDS = 2
EMB_STEPS = 3

NC = 2
NS = 16
NW = NC * NS
CHUNK = 128
NCHUNK = 40
EPW = CHUNK * NCHUNK
E_PAD = NW * EPW
N_PAD = 10240
ROWS_PER_SUB = N_PAD // NS
KBUF = 8

BLK_N = 256
TE = 1024


def _xh_body(x_ref, wt_ref, b_ref, o_ref, o2_ref):
    t = pl.program_id(0)
    xh = jnp.maximum(
        jnp.dot(x_ref[...], wt_ref[...], preferred_element_type=jnp.float32)
        + b_ref[...], 0.0)
    rows = t * BLK_N + lax.broadcasted_iota(jnp.int32, xh.shape, 0)
    xh = jnp.where(rows < N, xh, 0.0)
    o_ref[...] = xh
    o2_ref[...] = xh.astype(jnp.bfloat16)


def _msg_body(eat_ref, g_ref, n1w_ref, n1b_ref, n2w_ref, n2bm_ref, o_ref):
    h1 = jnp.maximum(
        jnp.dot(n1w_ref[...], eat_ref[...], preferred_element_type=jnp.float32)
        + n1b_ref[...], 0.0)
    wt = jnp.dot(n2w_ref[...], h1.astype(jnp.bfloat16),
                 preferred_element_type=jnp.float32)
    gt = g_ref[...].astype(jnp.float32).T
    acc = jnp.dot(n2bm_ref[...], gt, preferred_element_type=jnp.float32)
    for i in range(ATOM):
        acc = acc + wt[i * ATOM:(i + 1) * ATOM, :] * gt[i:i + 1, :]
    o_ref[...] = acc.T


def _gru_body(aggp_ref, degp_ref, h_ref, wih_ref, whh_ref, bih_ref, bhh_ref, o_ref, o2_ref):
    t = pl.program_id(0)
    aggp = aggp_ref[...]
    degp = degp_ref[...]
    agg = aggp[0] + aggp[1]
    deg = jnp.clip(degp[0] + degp[1], 1.0, None)
    m = jnp.maximum(agg / deg, 0.0)
    h = h_ref[...]
    gi = jnp.dot(m, wih_ref[...], preferred_element_type=jnp.float32) + bih_ref[...]
    gh = jnp.dot(h, whh_ref[...], preferred_element_type=jnp.float32) + bhh_ref[...]
    r = jax.nn.sigmoid(gi[:, :ATOM] + gh[:, :ATOM])
    z = jax.nn.sigmoid(gi[:, ATOM:2 * ATOM] + gh[:, ATOM:2 * ATOM])
    n = jnp.tanh(gi[:, 2 * ATOM:] + r * gh[:, 2 * ATOM:])
    hn = (1.0 - z) * n + z * h
    rows = t * BLK_N + lax.broadcasted_iota(jnp.int32, hn.shape, 0)
    hn = jnp.where(rows < N, hn, 0.0)
    o_ref[...] = hn
    o2_ref[...] = hn.astype(jnp.bfloat16)


def _s2s_body(xh_ref, batch_ref, wih_ref, whh_ref, bih_ref, bhh_ref,
              bng_ref, bnb_ref, bnrm_ref, bnrv_ref,
              m1w_ref, m1b_ref, m2w_ref, m2b_ref, pw_ref, pb_ref, o_ref):
    xh = xh_ref[...]
    b = batch_ref[...]
    gids = lax.broadcasted_iota(jnp.int32, (N_PAD, NGRAPH), 1)
    onehot_b = b == gids
    onehot = onehot_b.astype(jnp.float32)
    valid = b >= 0

    q_star = jnp.zeros((NGRAPH, 2 * ATOM), jnp.float32)
    hs = jnp.zeros((NGRAPH, ATOM), jnp.float32)
    cs = jnp.zeros((NGRAPH, ATOM), jnp.float32)
    for _ in range(EMB_STEPS):
        g = (jnp.dot(q_star, wih_ref[...], preferred_element_type=jnp.float32)
             + jnp.dot(hs, whh_ref[...], preferred_element_type=jnp.float32)
             + bih_ref[...] + bhh_ref[...])
        ig = jax.nn.sigmoid(g[:, :ATOM])
        fg = jax.nn.sigmoid(g[:, ATOM:2 * ATOM])
        gg = jnp.tanh(g[:, 2 * ATOM:3 * ATOM])
        og = jax.nn.sigmoid(g[:, 3 * ATOM:])
        cs = fg * cs + ig * gg
        hs = og * jnp.tanh(cs)
        q = hs
        d = jnp.dot(xh, q.T, preferred_element_type=jnp.float32)
        e = jnp.sum(d * onehot, axis=1, keepdims=True)
        mmax = jnp.max(jnp.where(onehot_b, e, -1e30), axis=0, keepdims=True)
        mmax_b = jnp.sum(onehot * mmax, axis=1, keepdims=True)
        a = jnp.exp(e - mmax_b)
        a = jnp.where(valid, a, 0.0)
        denom = jnp.sum(onehot * a, axis=0, keepdims=True)
        denom_b = jnp.sum(onehot * denom, axis=1, keepdims=True)
        a2 = jnp.where(valid, a / denom_b, 0.0)
        r = lax.dot_general(onehot, a2 * xh, (((0,), (0,)), ((), ())),
                            preferred_element_type=jnp.float32)
        q_star = jnp.concatenate([q, r], axis=1)

    qn = ((q_star - bnrm_ref[...]) * lax.rsqrt(bnrv_ref[...] + 1e-5)
          * bng_ref[...] + bnb_ref[...])
    h1 = jnp.maximum(jnp.dot(qn, m1w_ref[...], preferred_element_type=jnp.float32)
                     + m1b_ref[...], 0.0)
    h2 = jnp.maximum(jnp.dot(h1, m2w_ref[...], preferred_element_type=jnp.float32)
                     + m2b_ref[...], 0.0)
    o_ref[...] = jnp.dot(h2, pw_ref[...], preferred_element_type=jnp.float32) + pb_ref[...]


def _full(shape):
    return pl.BlockSpec(shape, lambda t: tuple(0 for _ in shape))


def _xh_call(xp, lin_wt, lin_b2):
    return pl.pallas_call(
        _xh_body,
        grid=(N_PAD // BLK_N,),
        in_specs=[
            pl.BlockSpec((BLK_N, NODE_DIM), lambda t: (t, 0)),
            _full((NODE_DIM, ATOM)),
            _full((1, ATOM)),
        ],
        out_specs=[pl.BlockSpec((BLK_N, ATOM), lambda t: (t, 0)),
                   pl.BlockSpec((BLK_N, ATOM), lambda t: (t, 0))],
        out_shape=[jax.ShapeDtypeStruct((N_PAD, ATOM), jnp.float32),
                   jax.ShapeDtypeStruct((N_PAD, ATOM), jnp.bfloat16)],
    )(xp, lin_wt, lin_b2)


def _msg_call(eat, g, n1_w, n1b2, n2_w, n2b2):
    return pl.pallas_call(
        _msg_body,
        grid=(E_PAD // TE,),
        in_specs=[
            pl.BlockSpec((EDGE_DIM, TE), lambda t: (0, t)),
            pl.BlockSpec((TE, ATOM), lambda t: (t, 0)),
            _full((CONV, EDGE_DIM)),
            _full((CONV, 1)),
            pl.BlockSpec((ATOM * ATOM, CONV), lambda t: (0, 0)),
            _full((ATOM, ATOM)),
        ],
        out_specs=pl.BlockSpec((TE, ATOM), lambda t: (t, 0)),
        out_shape=jax.ShapeDtypeStruct((E_PAD, ATOM), jnp.float32),
    )(eat, g, n1_w, n1b2, n2_w, n2b2)


def _gru_call(aggp, degp, h, wih_t, whh_t, bih2, bhh2):
    return pl.pallas_call(
        _gru_body,
        grid=(N_PAD // BLK_N,),
        in_specs=[
            pl.BlockSpec((NC, BLK_N, ATOM), lambda t: (0, t, 0)),
            pl.BlockSpec((NC, BLK_N, ATOM), lambda t: (0, t, 0)),
            pl.BlockSpec((BLK_N, ATOM), lambda t: (t, 0)),
            _full((ATOM, 3 * ATOM)),
            _full((ATOM, 3 * ATOM)),
            _full((1, 3 * ATOM)),
            _full((1, 3 * ATOM)),
        ],
        out_specs=[pl.BlockSpec((BLK_N, ATOM), lambda t: (t, 0)),
                   pl.BlockSpec((BLK_N, ATOM), lambda t: (t, 0))],
        out_shape=[jax.ShapeDtypeStruct((N_PAD, ATOM), jnp.float32),
                   jax.ShapeDtypeStruct((N_PAD, ATOM), jnp.bfloat16)],
    )(aggp, degp, h, wih_t, whh_t, bih2, bhh2)


def _s2s_call(xh, batch2d, wih_t, whh_t, bih2, bhh2, bng, bnb, bnrm, bnrv,
              m1w_t, m1b2, m2w_t, m2b2, pw_t, pb2):
    return pl.pallas_call(
        _s2s_body,
        out_shape=jax.ShapeDtypeStruct((NGRAPH, 1), jnp.float32),
    )(xh, batch2d, wih_t, whh_t, bih2, bhh2, bng, bnb, bnrm, bnrv,
      m1w_t, m1b2, m2w_t, m2b2, pw_t, pb2)


@functools.cache
def _sc_gather_kernel():
    mesh = plsc.VectorSubcoreMesh(core_axis_name="c", subcore_axis_name="s")
    return pl.kernel(
        _sc_gather_body, mesh=mesh,
        out_type=jax.ShapeDtypeStruct((E_PAD, ATOM), jnp.bfloat16),
        compiler_params=pltpu.CompilerParams(use_tc_tiling_on_sc=False),
        scratch_types=[
            pltpu.VMEM((NCHUNK, CHUNK), jnp.int32),
            pltpu.VMEM((2 * KBUF, CHUNK, ATOM), jnp.bfloat16),
            pltpu.SemaphoreType.DMA,
            pltpu.SemaphoreType.DMA,
        ],
    )


def _sc_gather_body(xh_hbm, src_hbm, out_hbm, idx_v, rows_v, sem_g, sem_w):
    c = lax.axis_index("c")
    s = lax.axis_index("s")
    wid = s * NC + c
    base = wid * EPW
    ngrp = NCHUNK // KBUF
    pltpu.sync_copy(src_hbm.at[wid], idx_v)

    def fire_gather(grp, half):
        for bidx in range(KBUF):
            j = grp * KBUF + bidx
            pltpu.make_async_copy(
                xh_hbm.at[idx_v.at[j]], rows_v.at[half * KBUF + bidx],
                sem_g).start()

    def wait_gather(grp, half):
        for bidx in range(KBUF):
            j = grp * KBUF + bidx
            pltpu.make_async_copy(
                xh_hbm.at[idx_v.at[j]], rows_v.at[half * KBUF + bidx],
                sem_g).wait()

    def fire_wb(grp, half, do):
        for bidx in range(KBUF):
            j = grp * KBUF + bidx
            cp = pltpu.make_async_copy(
                rows_v.at[half * KBUF + bidx],
                out_hbm.at[pl.ds(base + j * CHUNK, CHUNK)], sem_w)
            if do == "start":
                cp.start()
            else:
                cp.wait()

    fire_gather(0, 0)

    def outer(o, carry):
        half = lax.rem(o, 2)
        nhalf = lax.rem(o + 1, 2)

        @pl.when(o >= 1)
        def _():
            fire_wb(o - 1, nhalf, "wait")

        @pl.when(o + 1 < ngrp)
        def _():
            fire_gather(o + 1, nhalf)

        wait_gather(o, half)
        fire_wb(o, half, "start")
        return carry

    lax.fori_loop(0, ngrp, outer, 0)
    fire_wb(ngrp - 1, (ngrp - 1) % 2, "wait")


@functools.cache
def _sc_scatter_kernel():
    mesh = plsc.VectorSubcoreMesh(core_axis_name="c", subcore_axis_name="s")
    return pl.kernel(
        _sc_scatter_body, mesh=mesh,
        out_type=jax.ShapeDtypeStruct((NC, N_PAD, ATOM), jnp.float32),
        compiler_params=pltpu.CompilerParams(use_tc_tiling_on_sc=False),
        scratch_types=[
            pltpu.VMEM((NCHUNK, CHUNK), jnp.int32),
            pltpu.VMEM((2 * KBUF, CHUNK, ATOM), jnp.float32),
            pltpu.VMEM_SHARED((N_PAD, ATOM), jnp.float32),
            pltpu.SemaphoreType.DMA,
        ],
    )


def _sc_scatter_body(msg_hbm, dst_hbm, zeros_hbm, out_hbm, idx_v, buf_v, acc_sh, sem_l):
    c = lax.axis_index("c")
    s = lax.axis_index("s")
    wid = s * NC + c
    base = wid * EPW
    pltpu.sync_copy(dst_hbm.at[wid], idx_v)
    pltpu.sync_copy(zeros_hbm.at[pl.ds(s * ROWS_PER_SUB, ROWS_PER_SUB)],
                    acc_sh.at[pl.ds(s * ROWS_PER_SUB, ROWS_PER_SUB)])
    plsc.subcore_barrier()

    ngrp = NCHUNK // KBUF

    def fire_load(grp, half, do):
        for bidx in range(KBUF):
            j = grp * KBUF + bidx
            cp = pltpu.make_async_copy(
                msg_hbm.at[pl.ds(base + j * CHUNK, CHUNK)],
                buf_v.at[half * KBUF + bidx], sem_l)
            if do == "start":
                cp.start()
            else:
                cp.wait()

    fire_load(0, 0, "start")

    def outer(o, carry):
        half = lax.rem(o, 2)
        nhalf = lax.rem(o + 1, 2)

        @pl.when(o + 1 < ngrp)
        def _():
            fire_load(o + 1, nhalf, "start")

        fire_load(o, half, "wait")
        for bidx in range(KBUF):
            j = o * KBUF + bidx
            pltpu.sync_copy(buf_v.at[half * KBUF + bidx],
                            acc_sh.at[idx_v.at[j]], add=True)
        return carry

    lax.fori_loop(0, ngrp, outer, 0)
    plsc.subcore_barrier()
    pltpu.sync_copy(acc_sh.at[pl.ds(s * ROWS_PER_SUB, ROWS_PER_SUB)],
                    out_hbm.at[c, pl.ds(s * ROWS_PER_SUB, ROWS_PER_SUB)])


@functools.cache
def _sc_deg_kernel():
    mesh = plsc.VectorSubcoreMesh(core_axis_name="c", subcore_axis_name="s")
    return pl.kernel(
        _sc_deg_body, mesh=mesh,
        out_type=jax.ShapeDtypeStruct((NC, N_PAD, ATOM), jnp.float32),
        compiler_params=pltpu.CompilerParams(use_tc_tiling_on_sc=False),
        scratch_types=[
            pltpu.VMEM((NCHUNK, CHUNK), jnp.int32),
            pltpu.VMEM((CHUNK, ATOM), jnp.float32),
            pltpu.VMEM_SHARED((N_PAD, ATOM), jnp.float32),
        ],
    )


def _sc_deg_body(dst_hbm, ones_hbm, zeros_hbm, out_hbm, idx_v, ones_v, acc_sh):
    c = lax.axis_index("c")
    s = lax.axis_index("s")
    wid = s * NC + c
    pltpu.sync_copy(dst_hbm.at[wid], idx_v)
    pltpu.sync_copy(ones_hbm, ones_v)
    pltpu.sync_copy(zeros_hbm.at[pl.ds(s * ROWS_PER_SUB, ROWS_PER_SUB)],
                    acc_sh.at[pl.ds(s * ROWS_PER_SUB, ROWS_PER_SUB)])
    plsc.subcore_barrier()

    def body(j, carry):
        pltpu.sync_copy(ones_v, acc_sh.at[idx_v.at[j]], add=True)
        return carry

    lax.fori_loop(0, NCHUNK, body, 0)
    plsc.subcore_barrier()
    pltpu.sync_copy(acc_sh.at[pl.ds(s * ROWS_PER_SUB, ROWS_PER_SUB)],
                    out_hbm.at[c, pl.ds(s * ROWS_PER_SUB, ROWS_PER_SUB)])


def kernel(x, edge_attr, edge_index, batch, lin_w, lin_b, n1_w, n1_b, n2_w, n2_b,
           gru_wih, gru_whh, gru_bih, gru_bhh, lstm_wih, lstm_whh, lstm_bih,
           lstm_bhh, bn_g, bn_b, bn_rm, bn_rv, m1_w, m1_b, m2_w, m2_b, p_w, p_b):
    xp = jnp.pad(x, ((0, N_PAD - N), (0, 0)))
    eat = jnp.pad(edge_attr, ((0, E_PAD - E), (0, 0))).T
    src = jnp.pad(edge_index[0], (0, E_PAD - E)).reshape(NW, NCHUNK, CHUNK)
    dst = jnp.pad(edge_index[1], (0, E_PAD - E),
                  constant_values=N).reshape(NW, NCHUNK, CHUNK)
    batch2d = jnp.pad(batch, (0, N_PAD - N), constant_values=-1).reshape(N_PAD, 1)
    zeros_n = jnp.zeros((N_PAD, ATOM), jnp.float32)
    ones_c = jnp.ones((CHUNK, ATOM), jnp.float32)

    xh, xh_bf = _xh_call(xp, lin_w.T, lin_b.reshape(1, ATOM))
    degp = _sc_deg_kernel()(dst, ones_c, zeros_n)

    h = xh
    n1b2 = n1_b.reshape(CONV, 1)
    n2w_bf = n2_w.astype(jnp.bfloat16)
    n2b2 = n2_b.reshape(ATOM, ATOM).T
    gru_wih_t = gru_wih.T
    gru_whh_t = gru_whh.T
    gru_bih2 = gru_bih.reshape(1, 3 * ATOM)
    gru_bhh2 = gru_bhh.reshape(1, 3 * ATOM)
    for _ in range(NUM_EMBEDS):
        g = _sc_gather_kernel()(xh_bf, src)
        msg = _msg_call(eat, g, n1_w, n1b2, n2w_bf, n2b2)
        aggp = _sc_scatter_kernel()(msg, dst, zeros_n)
        h, xh_bf = _gru_call(aggp, degp, h, gru_wih_t, gru_whh_t, gru_bih2, gru_bhh2)
        xh = h

    out = _s2s_call(
        xh, batch2d, lstm_wih.T, lstm_whh.T,
        lstm_bih.reshape(1, 4 * ATOM), lstm_bhh.reshape(1, 4 * ATOM),
        bn_g.reshape(1, 2 * ATOM), bn_b.reshape(1, 2 * ATOM),
        bn_rm.reshape(1, 2 * ATOM), bn_rv.reshape(1, 2 * ATOM),
        m1_w.T, m1_b.reshape(1, -1), m2_w.T, m2_b.reshape(1, -1),
        p_w.T, p_b.reshape(1, 1))
    return out


# speedup vs baseline: 1.0443x; 1.0443x over previous
import functools

import jax
import jax.numpy as jnp
from jax import lax
from jax.experimental import pallas as pl
from jax.experimental.pallas import tpu as pltpu
from jax.experimental.pallas import tpu_sc as plsc

N = 10000
E = 160000
NODE_DIM = 128
EDGE_DIM = 16
ATOM = 32
CONV = 128
NGRAPH = 64
NUM_EMBEDS = 2
EMB_STEPS = 3

NC = 2
NS = 16
NW = NC * NS
CHUNK = 128
NCHUNK = 40
EPW = CHUNK * NCHUNK
E_PAD = NW * EPW
N_PAD = 10240
ROWS_PER_SUB = N_PAD // NS
KBUF = 8

BLK_N = 256
TE = 1024


def _xh_body(x_ref, wt_ref, b_ref, o_ref):
    t = pl.program_id(0)
    xh = jnp.maximum(
        jnp.dot(x_ref[...], wt_ref[...], preferred_element_type=jnp.float32)
        + b_ref[...], 0.0)
    rows = t * BLK_N + lax.broadcasted_iota(jnp.int32, xh.shape, 0)
    o_ref[...] = jnp.where(rows < N, xh, 0.0)


def _msg_body(eat_ref, g_ref, n1w_ref, n1b_ref, n2w_ref, n2b_ref, o_ref):
    h1 = jnp.maximum(
        jnp.dot(n1w_ref[...], eat_ref[...], preferred_element_type=jnp.float32)
        + n1b_ref[...], 0.0)
    wt = jnp.dot(n2w_ref[...], h1.astype(jnp.bfloat16),
                 preferred_element_type=jnp.float32) + n2b_ref[...]
    gt = g_ref[...].T
    acc = jnp.zeros((ATOM, TE), jnp.float32)
    for i in range(ATOM):
        acc = acc + wt[i * ATOM:(i + 1) * ATOM, :] * gt[i:i + 1, :]
    o_ref[...] = acc.T


def _gru_body(aggp_ref, degp_ref, h_ref, wih_ref, whh_ref, bih_ref, bhh_ref, o_ref):
    t = pl.program_id(0)
    aggp = aggp_ref[...]
    degp = degp_ref[...]
    agg = aggp[0] + aggp[1]
    deg = jnp.clip(degp[0] + degp[1], 1.0, None)
    m = jnp.maximum(agg / deg, 0.0)
    h = h_ref[...]
    gi = jnp.dot(m, wih_ref[...], preferred_element_type=jnp.float32) + bih_ref[...]
    gh = jnp.dot(h, whh_ref[...], preferred_element_type=jnp.float32) + bhh_ref[...]
    r = jax.nn.sigmoid(gi[:, :ATOM] + gh[:, :ATOM])
    z = jax.nn.sigmoid(gi[:, ATOM:2 * ATOM] + gh[:, ATOM:2 * ATOM])
    n = jnp.tanh(gi[:, 2 * ATOM:] + r * gh[:, 2 * ATOM:])
    hn = (1.0 - z) * n + z * h
    rows = t * BLK_N + lax.broadcasted_iota(jnp.int32, hn.shape, 0)
    o_ref[...] = jnp.where(rows < N, hn, 0.0)


def _s2s_body(xh_ref, batch_ref, wih_ref, whh_ref, bih_ref, bhh_ref,
              bng_ref, bnb_ref, bnrm_ref, bnrv_ref,
              m1w_ref, m1b_ref, m2w_ref, m2b_ref, pw_ref, pb_ref, o_ref):
    xh = xh_ref[...]
    b = batch_ref[...]
    gids = lax.broadcasted_iota(jnp.int32, (N_PAD, NGRAPH), 1)
    onehot_b = b == gids
    onehot = onehot_b.astype(jnp.float32)
    valid = b >= 0

    q_star = jnp.zeros((NGRAPH, 2 * ATOM), jnp.float32)
    hs = jnp.zeros((NGRAPH, ATOM), jnp.float32)
    cs = jnp.zeros((NGRAPH, ATOM), jnp.float32)
    for _ in range(EMB_STEPS):
        g = (jnp.dot(q_star, wih_ref[...], preferred_element_type=jnp.float32)
             + jnp.dot(hs, whh_ref[...], preferred_element_type=jnp.float32)
             + bih_ref[...] + bhh_ref[...])
        ig = jax.nn.sigmoid(g[:, :ATOM])
        fg = jax.nn.sigmoid(g[:, ATOM:2 * ATOM])
        gg = jnp.tanh(g[:, 2 * ATOM:3 * ATOM])
        og = jax.nn.sigmoid(g[:, 3 * ATOM:])
        cs = fg * cs + ig * gg
        hs = og * jnp.tanh(cs)
        q = hs
        d = jnp.dot(xh, q.T, preferred_element_type=jnp.float32)
        e = jnp.sum(d * onehot, axis=1, keepdims=True)
        mmax = jnp.max(jnp.where(onehot_b, e, -1e30), axis=0, keepdims=True)
        mmax_b = jnp.sum(onehot * mmax, axis=1, keepdims=True)
        a = jnp.exp(e - mmax_b)
        a = jnp.where(valid, a, 0.0)
        denom = jnp.sum(onehot * a, axis=0, keepdims=True)
        denom_b = jnp.sum(onehot * denom, axis=1, keepdims=True)
        a2 = jnp.where(valid, a / denom_b, 0.0)
        r = lax.dot_general(onehot, a2 * xh, (((0,), (0,)), ((), ())),
                            preferred_element_type=jnp.float32)
        q_star = jnp.concatenate([q, r], axis=1)

    qn = ((q_star - bnrm_ref[...]) * lax.rsqrt(bnrv_ref[...] + 1e-5)
          * bng_ref[...] + bnb_ref[...])
    h1 = jnp.maximum(jnp.dot(qn, m1w_ref[...], preferred_element_type=jnp.float32)
                     + m1b_ref[...], 0.0)
    h2 = jnp.maximum(jnp.dot(h1, m2w_ref[...], preferred_element_type=jnp.float32)
                     + m2b_ref[...], 0.0)
    o_ref[...] = jnp.dot(h2, pw_ref[...], preferred_element_type=jnp.float32) + pb_ref[...]


def _full(shape):
    return pl.BlockSpec(shape, lambda t: tuple(0 for _ in shape))


def _xh_call(xp, lin_wt, lin_b2):
    return pl.pallas_call(
        _xh_body,
        grid=(N_PAD // BLK_N,),
        in_specs=[
            pl.BlockSpec((BLK_N, NODE_DIM), lambda t: (t, 0)),
            _full((NODE_DIM, ATOM)),
            _full((1, ATOM)),
        ],
        out_specs=pl.BlockSpec((BLK_N, ATOM), lambda t: (t, 0)),
        out_shape=jax.ShapeDtypeStruct((N_PAD, ATOM), jnp.float32),
    )(xp, lin_wt, lin_b2)


def _msg_call(eat, g, n1_w, n1b2, n2_w, n2b2):
    return pl.pallas_call(
        _msg_body,
        grid=(E_PAD // TE,),
        in_specs=[
            pl.BlockSpec((EDGE_DIM, TE), lambda t: (0, t)),
            pl.BlockSpec((TE, ATOM), lambda t: (t, 0)),
            _full((CONV, EDGE_DIM)),
            _full((CONV, 1)),
            pl.BlockSpec((ATOM * ATOM, CONV), lambda t: (0, 0)),
            _full((ATOM * ATOM, 1)),
        ],
        out_specs=pl.BlockSpec((TE, ATOM), lambda t: (t, 0)),
        out_shape=jax.ShapeDtypeStruct((E_PAD, ATOM), jnp.float32),
    )(eat, g, n1_w, n1b2, n2_w, n2b2)


def _gru_call(aggp, degp, h, wih_t, whh_t, bih2, bhh2):
    return pl.pallas_call(
        _gru_body,
        grid=(N_PAD // BLK_N,),
        in_specs=[
            pl.BlockSpec((NC, BLK_N, ATOM), lambda t: (0, t, 0)),
            pl.BlockSpec((NC, BLK_N, ATOM), lambda t: (0, t, 0)),
            pl.BlockSpec((BLK_N, ATOM), lambda t: (t, 0)),
            _full((ATOM, 3 * ATOM)),
            _full((ATOM, 3 * ATOM)),
            _full((1, 3 * ATOM)),
            _full((1, 3 * ATOM)),
        ],
        out_specs=pl.BlockSpec((BLK_N, ATOM), lambda t: (t, 0)),
        out_shape=jax.ShapeDtypeStruct((N_PAD, ATOM), jnp.float32),
    )(aggp, degp, h, wih_t, whh_t, bih2, bhh2)


def _s2s_call(xh, batch2d, wih_t, whh_t, bih2, bhh2, bng, bnb, bnrm, bnrv,
              m1w_t, m1b2, m2w_t, m2b2, pw_t, pb2):
    return pl.pallas_call(
        _s2s_body,
        out_shape=jax.ShapeDtypeStruct((NGRAPH, 1), jnp.float32),
    )(xh, batch2d, wih_t, whh_t, bih2, bhh2, bng, bnb, bnrm, bnrv,
      m1w_t, m1b2, m2w_t, m2b2, pw_t, pb2)


@functools.cache
def _sc_gather_kernel():
    mesh = plsc.VectorSubcoreMesh(core_axis_name="c", subcore_axis_name="s")
    return pl.kernel(
        _sc_gather_body, mesh=mesh,
        out_type=jax.ShapeDtypeStruct((E_PAD, ATOM), jnp.float32),
        compiler_params=pltpu.CompilerParams(use_tc_tiling_on_sc=False),
        scratch_types=[
            pltpu.VMEM((NCHUNK, CHUNK), jnp.int32),
            pltpu.VMEM((2 * KBUF, CHUNK, ATOM), jnp.float32),
            pltpu.SemaphoreType.DMA,
            pltpu.SemaphoreType.DMA,
        ],
    )


def _sc_gather_body(xh_hbm, src_hbm, out_hbm, idx_v, rows_v, sem_g, sem_w):
    c = lax.axis_index("c")
    s = lax.axis_index("s")
    wid = s * NC + c
    base = wid * EPW
    ngrp = NCHUNK // KBUF
    pltpu.sync_copy(src_hbm.at[wid], idx_v)

    def fire_gather(grp, half):
        for bidx in range(KBUF):
            j = grp * KBUF + bidx
            pltpu.make_async_copy(
                xh_hbm.at[idx_v.at[j]], rows_v.at[half * KBUF + bidx],
                sem_g).start()

    def wait_gather(grp, half):
        for bidx in range(KBUF):
            j = grp * KBUF + bidx
            pltpu.make_async_copy(
                xh_hbm.at[idx_v.at[j]], rows_v.at[half * KBUF + bidx],
                sem_g).wait()

    def fire_wb(grp, half, do):
        for bidx in range(KBUF):
            j = grp * KBUF + bidx
            cp = pltpu.make_async_copy(
                rows_v.at[half * KBUF + bidx],
                out_hbm.at[pl.ds(base + j * CHUNK, CHUNK)], sem_w)
            if do == "start":
                cp.start()
            else:
                cp.wait()

    fire_gather(0, 0)

    def outer(o, carry):
        half = lax.rem(o, 2)
        nhalf = lax.rem(o + 1, 2)

        @pl.when(o >= 1)
        def _():
            fire_wb(o - 1, nhalf, "wait")

        @pl.when(o + 1 < ngrp)
        def _():
            fire_gather(o + 1, nhalf)

        wait_gather(o, half)
        fire_wb(o, half, "start")
        return carry

    lax.fori_loop(0, ngrp, outer, 0)
    fire_wb(ngrp - 1, (ngrp - 1) % 2, "wait")


@functools.cache
def _sc_scatter_kernel():
    mesh = plsc.VectorSubcoreMesh(core_axis_name="c", subcore_axis_name="s")
    return pl.kernel(
        _sc_scatter_body, mesh=mesh,
        out_type=jax.ShapeDtypeStruct((NC, N_PAD, ATOM), jnp.float32),
        compiler_params=pltpu.CompilerParams(use_tc_tiling_on_sc=False),
        scratch_types=[
            pltpu.VMEM((NCHUNK, CHUNK), jnp.int32),
            pltpu.VMEM((2 * KBUF, CHUNK, ATOM), jnp.float32),
            pltpu.VMEM_SHARED((N_PAD, ATOM), jnp.float32),
            pltpu.SemaphoreType.DMA,
        ],
    )


def _sc_scatter_body(msg_hbm, dst_hbm, zeros_hbm, out_hbm, idx_v, buf_v, acc_sh, sem_l):
    c = lax.axis_index("c")
    s = lax.axis_index("s")
    wid = s * NC + c
    base = wid * EPW
    pltpu.sync_copy(dst_hbm.at[wid], idx_v)
    pltpu.sync_copy(zeros_hbm.at[pl.ds(s * ROWS_PER_SUB, ROWS_PER_SUB)],
                    acc_sh.at[pl.ds(s * ROWS_PER_SUB, ROWS_PER_SUB)])
    plsc.subcore_barrier()

    ngrp = NCHUNK // KBUF

    def fire_load(grp, half, do):
        for bidx in range(KBUF):
            j = grp * KBUF + bidx
            cp = pltpu.make_async_copy(
                msg_hbm.at[pl.ds(base + j * CHUNK, CHUNK)],
                buf_v.at[half * KBUF + bidx], sem_l)
            if do == "start":
                cp.start()
            else:
                cp.wait()

    fire_load(0, 0, "start")

    def outer(o, carry):
        half = lax.rem(o, 2)
        nhalf = lax.rem(o + 1, 2)

        @pl.when(o + 1 < ngrp)
        def _():
            fire_load(o + 1, nhalf, "start")

        fire_load(o, half, "wait")
        for bidx in range(KBUF):
            j = o * KBUF + bidx
            pltpu.sync_copy(buf_v.at[half * KBUF + bidx],
                            acc_sh.at[idx_v.at[j]], add=True)
        return carry

    lax.fori_loop(0, ngrp, outer, 0)
    plsc.subcore_barrier()
    pltpu.sync_copy(acc_sh.at[pl.ds(s * ROWS_PER_SUB, ROWS_PER_SUB)],
                    out_hbm.at[c, pl.ds(s * ROWS_PER_SUB, ROWS_PER_SUB)])


@functools.cache
def _sc_deg_kernel():
    mesh = plsc.VectorSubcoreMesh(core_axis_name="c", subcore_axis_name="s")
    return pl.kernel(
        _sc_deg_body, mesh=mesh,
        out_type=jax.ShapeDtypeStruct((NC, N_PAD, ATOM), jnp.float32),
        compiler_params=pltpu.CompilerParams(use_tc_tiling_on_sc=False),
        scratch_types=[
            pltpu.VMEM((NCHUNK, CHUNK), jnp.int32),
            pltpu.VMEM((CHUNK, ATOM), jnp.float32),
            pltpu.VMEM_SHARED((N_PAD, ATOM), jnp.float32),
        ],
    )


def _sc_deg_body(dst_hbm, ones_hbm, zeros_hbm, out_hbm, idx_v, ones_v, acc_sh):
    c = lax.axis_index("c")
    s = lax.axis_index("s")
    wid = s * NC + c
    pltpu.sync_copy(dst_hbm.at[wid], idx_v)
    pltpu.sync_copy(ones_hbm, ones_v)
    pltpu.sync_copy(zeros_hbm.at[pl.ds(s * ROWS_PER_SUB, ROWS_PER_SUB)],
                    acc_sh.at[pl.ds(s * ROWS_PER_SUB, ROWS_PER_SUB)])
    plsc.subcore_barrier()

    def body(j, carry):
        pltpu.sync_copy(ones_v, acc_sh.at[idx_v.at[j]], add=True)
        return carry

    lax.fori_loop(0, NCHUNK, body, 0)
    plsc.subcore_barrier()
    pltpu.sync_copy(acc_sh.at[pl.ds(s * ROWS_PER_SUB, ROWS_PER_SUB)],
                    out_hbm.at[c, pl.ds(s * ROWS_PER_SUB, ROWS_PER_SUB)])


def kernel(x, edge_attr, edge_index, batch, lin_w, lin_b, n1_w, n1_b, n2_w, n2_b,
           gru_wih, gru_whh, gru_bih, gru_bhh, lstm_wih, lstm_whh, lstm_bih,
           lstm_bhh, bn_g, bn_b, bn_rm, bn_rv, m1_w, m1_b, m2_w, m2_b, p_w, p_b):
    xp = jnp.pad(x, ((0, N_PAD - N), (0, 0)))
    eat = jnp.pad(edge_attr, ((0, E_PAD - E), (0, 0))).T
    src = jnp.pad(edge_index[0], (0, E_PAD - E)).reshape(NW, NCHUNK, CHUNK)
    dst = jnp.pad(edge_index[1], (0, E_PAD - E),
                  constant_values=N).reshape(NW, NCHUNK, CHUNK)
    batch2d = jnp.pad(batch, (0, N_PAD - N), constant_values=-1).reshape(N_PAD, 1)
    zeros_n = jnp.zeros((N_PAD, ATOM), jnp.float32)
    ones_c = jnp.ones((CHUNK, ATOM), jnp.float32)

    xh = _xh_call(xp, lin_w.T, lin_b.reshape(1, ATOM))
    degp = _sc_deg_kernel()(dst, ones_c, zeros_n)

    h = xh
    n1b2 = n1_b.reshape(CONV, 1)
    n2w_bf = n2_w.astype(jnp.bfloat16)
    n2b2 = n2_b.reshape(ATOM * ATOM, 1)
    gru_wih_t = gru_wih.T
    gru_whh_t = gru_whh.T
    gru_bih2 = gru_bih.reshape(1, 3 * ATOM)
    gru_bhh2 = gru_bhh.reshape(1, 3 * ATOM)
    for _ in range(NUM_EMBEDS):
        g = _sc_gather_kernel()(xh, src)
        msg = _msg_call(eat, g, n1_w, n1b2, n2w_bf, n2b2)
        aggp = _sc_scatter_kernel()(msg, dst, zeros_n)
        h = _gru_call(aggp, degp, h, gru_wih_t, gru_whh_t, gru_bih2, gru_bhh2)
        xh = h

    out = _s2s_call(
        xh, batch2d, lstm_wih.T, lstm_whh.T,
        lstm_bih.reshape(1, 4 * ATOM), lstm_bhh.reshape(1, 4 * ATOM),
        bn_g.reshape(1, 2 * ATOM), bn_b.reshape(1, 2 * ATOM),
        bn_rm.reshape(1, 2 * ATOM), bn_rv.reshape(1, 2 * ATOM),
        m1_w.T, m1_b.reshape(1, -1), m2_w.T, m2_b.reshape(1, -1),
        p_w.T, p_b.reshape(1, 1))
    return out
